# baseline (speedup 1.0000x reference)
import dataclasses
import functools

import jax
import jax.numpy as jnp
from jax import lax
from jax.experimental import pallas as pl
from jax.experimental.pallas import tpu as pltpu
from jax.experimental.pallas import tpu_sc as plsc

D = 256
DP = D // 2
RW = DP + 1
L = 16
NC, NS = 2, 16
NWORK = NC * NS
W = 128
PC = 8


def _pack_bf16(x):
    lo = lax.bitcast_convert_type(
        x[:, 0::2].astype(jnp.bfloat16), jnp.uint16).astype(jnp.uint32)
    hi = lax.bitcast_convert_type(
        x[:, 1::2].astype(jnp.bfloat16), jnp.uint16).astype(jnp.uint32)
    w = lax.bitcast_convert_type(lo | (hi << 16), jnp.int32)
    return jnp.pad(w, ((0, 0), (0, RW - w.shape[1])))


def kernel(x_user, x_job, edge_label_index):
    E = edge_label_index.shape[1]
    n_win_total = E // W
    mesh = plsc.VectorSubcoreMesh(core_axis_name="c", subcore_axis_name="s")
    cp = pltpu.CompilerParams()
    if "needs_layout_passes" in pltpu.CompilerParams.__dataclass_fields__:
        cp = dataclasses.replace(cp, needs_layout_passes=False)
    if "use_tc_tiling_on_sc" in pltpu.CompilerParams.__dataclass_fields__:
        cp = dataclasses.replace(cp, use_tc_tiling_on_sc=False)

    @functools.partial(
        pl.kernel,
        out_type=jax.ShapeDtypeStruct((E,), jnp.float32),
        mesh=mesh,
        compiler_params=cp,
        scratch_types=[
            pltpu.VMEM((W,), jnp.int32),
            pltpu.VMEM((W,), jnp.int32),
            pltpu.VMEM((W, RW), jnp.int32),
            pltpu.VMEM((W, RW), jnp.int32),
            pltpu.VMEM((W,), jnp.float32),
            pltpu.SemaphoreType.DMA,
            pltpu.SemaphoreType.DMA,
        ],
    )
    def sc_kernel(xu_hbm, xj_hbm, eu_hbm, ej_hbm, out_hbm,
                  iu_v, ij_v, ru_v, rj_v, out_v, sem_u, sem_j):
        wid = lax.axis_index("s") * NC + lax.axis_index("c")
        nw = (n_win_total + NWORK - 1 - wid) // NWORK

        @pl.loop(0, nw)
        def _(g):
            base = (wid + g * NWORK) * W
            pltpu.sync_copy(eu_hbm.at[pl.ds(base, W)], iu_v)
            pltpu.sync_copy(ej_hbm.at[pl.ds(base, W)], ij_v)
            cu = pltpu.async_copy(xu_hbm.at[iu_v], ru_v, sem_u)
            cj = pltpu.async_copy(xj_hbm.at[ij_v], rj_v, sem_j)
            cu.wait()
            cj.wait()

            @pl.loop(0, W // L)
            def _(gi):
                e0 = gi * L
                e_ids = e0 + lax.iota(jnp.int32, L)
                zero = jnp.zeros((L,), jnp.float32)

                def dim_body(t, accs):
                    accs = list(accs)
                    p0 = t * PC
                    for pp in range(PC):
                        pvec = jnp.zeros((L,), jnp.int32) + (p0 + pp)
                        uw = plsc.load_gather(ru_v, [e_ids, pvec])
                        vw = plsc.load_gather(rj_v, [e_ids, pvec])
                        ua, ub = plsc.unpack(
                            plsc.bitcast(uw, jnp.bfloat16),
                            format=plsc.PackFormat.INTERLEAVED)
                        va, vb = plsc.unpack(
                            plsc.bitcast(vw, jnp.bfloat16),
                            format=plsc.PackFormat.INTERLEAVED)
                        accs[(2 * pp) % 4] = accs[(2 * pp) % 4] + ua * va
                        accs[(2 * pp + 1) % 4] = accs[(2 * pp + 1) % 4] + ub * vb
                    return tuple(accs)

                a0, a1, a2, a3 = lax.fori_loop(
                    0, DP // PC, dim_body, (zero, zero, zero, zero))
                out_v[pl.ds(e0, L)] = (a0 + a1) + (a2 + a3)

            pltpu.sync_copy(out_v, out_hbm.at[pl.ds(base, W)])

    return sc_kernel(_pack_bf16(x_user), _pack_bf16(x_job),
                     edge_label_index[0], edge_label_index[1])


# speedup vs baseline: 2.8736x; 2.8736x over previous
import dataclasses
import functools

import jax
import jax.numpy as jnp
from jax import lax
from jax.experimental import pallas as pl
from jax.experimental.pallas import tpu as pltpu
from jax.experimental.pallas import tpu_sc as plsc

D = 256
DP = D // 2
RW = DP + 1
L = 16
NC, NS = 2, 16
NWORK = NC * NS
W = 128
PC = 8


def _pack_bf16(x):
    n, d = x.shape
    h = d // 2
    lo = lax.bitcast_convert_type(
        x[:, :h].astype(jnp.bfloat16), jnp.uint16).astype(jnp.uint32)
    hi = lax.bitcast_convert_type(
        x[:, h:].astype(jnp.bfloat16), jnp.uint16).astype(jnp.uint32)
    w = lax.bitcast_convert_type(lo | (hi << 16), jnp.int32)
    return jnp.pad(w, ((0, 0), (0, RW - h)))


def kernel(x_user, x_job, edge_label_index):
    E = edge_label_index.shape[1]
    n_win_total = E // W
    n_per_worker = -(-n_win_total // NWORK)
    n_pairs = -(-n_per_worker // 2)
    mesh = plsc.VectorSubcoreMesh(core_axis_name="c", subcore_axis_name="s")
    cp = pltpu.CompilerParams()
    if "needs_layout_passes" in pltpu.CompilerParams.__dataclass_fields__:
        cp = dataclasses.replace(cp, needs_layout_passes=False)
    if "use_tc_tiling_on_sc" in pltpu.CompilerParams.__dataclass_fields__:
        cp = dataclasses.replace(cp, use_tc_tiling_on_sc=False)

    @functools.partial(
        pl.kernel,
        out_type=jax.ShapeDtypeStruct((E,), jnp.float32),
        mesh=mesh,
        compiler_params=cp,
        scratch_types=[
            pltpu.VMEM((W,), jnp.int32),
            pltpu.VMEM((W,), jnp.int32),
            pltpu.VMEM((W,), jnp.int32),
            pltpu.VMEM((W,), jnp.int32),
            pltpu.VMEM((W, RW), jnp.int32),
            pltpu.VMEM((W, RW), jnp.int32),
            pltpu.VMEM((W, RW), jnp.int32),
            pltpu.VMEM((W, RW), jnp.int32),
            pltpu.VMEM((W,), jnp.float32),
            pltpu.SemaphoreType.DMA,
            pltpu.SemaphoreType.DMA,
            pltpu.SemaphoreType.DMA,
            pltpu.SemaphoreType.DMA,
        ],
    )
    def sc_kernel(xu_hbm, xj_hbm, eu_hbm, ej_hbm, out_hbm,
                  iu0, iu1, ij0, ij1, ru0, ru1, rj0, rj1, out_v,
                  su0, su1, sj0, sj1):
        wid = lax.axis_index("s") * NC + lax.axis_index("c")
        ius, ijs, rus, rjs = (iu0, iu1), (ij0, ij1), (ru0, ru1), (rj0, rj1)
        sus, sjs = (su0, su1), (sj0, sj1)

        def win_base(k):
            win = wid + k * NWORK
            valid = win < n_win_total
            return jnp.where(valid, win * W, 0), valid

        def fetch(k, s):
            base, _ = win_base(k)
            pltpu.sync_copy(eu_hbm.at[pl.ds(base, W)], ius[s])
            pltpu.sync_copy(ej_hbm.at[pl.ds(base, W)], ijs[s])
            pltpu.async_copy(xu_hbm.at[ius[s]], rus[s], sus[s])
            pltpu.async_copy(xj_hbm.at[ijs[s]], rjs[s], sjs[s])

        def wait_rows(s):
            pltpu.make_async_copy(xu_hbm.at[ius[s]], rus[s], sus[s]).wait()
            pltpu.make_async_copy(xj_hbm.at[ijs[s]], rjs[s], sjs[s]).wait()

        def compute_store(k, s):
            ru_v, rj_v = rus[s], rjs[s]

            @pl.loop(0, W // L)
            def _(gi):
                e0 = gi * L
                e_ids = e0 + lax.iota(jnp.int32, L)
                zero = jnp.zeros((L,), jnp.float32)

                def dim_body(t, accs):
                    accs = list(accs)
                    p0 = t * PC
                    for pp in range(PC):
                        pvec = jnp.zeros((L,), jnp.int32) + (p0 + pp)
                        uw = plsc.load_gather(ru_v, [e_ids, pvec])
                        vw = plsc.load_gather(rj_v, [e_ids, pvec])
                        ua, ub = plsc.unpack(
                            plsc.bitcast(uw, jnp.bfloat16),
                            format=plsc.PackFormat.INTERLEAVED)
                        va, vb = plsc.unpack(
                            plsc.bitcast(vw, jnp.bfloat16),
                            format=plsc.PackFormat.INTERLEAVED)
                        accs[(2 * pp) % 4] = accs[(2 * pp) % 4] + ua * va
                        accs[(2 * pp + 1) % 4] = accs[(2 * pp + 1) % 4] + ub * vb
                    return tuple(accs)

                a0, a1, a2, a3 = lax.fori_loop(
                    0, DP // PC, dim_body, (zero, zero, zero, zero))
                out_v[pl.ds(e0, L)] = (a0 + a1) + (a2 + a3)

            base, valid = win_base(k)

            @pl.when(valid)
            def _():
                pltpu.sync_copy(out_v, out_hbm.at[pl.ds(base, W)])

        fetch(0, 0)

        @pl.loop(0, n_pairs)
        def _(i):
            k0 = 2 * i
            fetch(k0 + 1, 1)
            wait_rows(0)
            compute_store(k0, 0)
            fetch(k0 + 2, 0)
            wait_rows(1)
            compute_store(k0 + 1, 1)

        wait_rows(0)

    return sc_kernel(_pack_bf16(x_user), _pack_bf16(x_job),
                     edge_label_index[0], edge_label_index[1])


# speedup vs baseline: 3.4200x; 1.1901x over previous
import dataclasses
import functools

import jax
import jax.numpy as jnp
from jax import lax
from jax.experimental import pallas as pl
from jax.experimental.pallas import tpu as pltpu
from jax.experimental.pallas import tpu_sc as plsc

D = 256
DP = D // 2
RW = DP + 1
L = 16
NC, NS = 2, 16
NWORK = NC * NS
W = 128
PC = 8


def _pack_bf16(x):
    n, d = x.shape
    h = d // 2
    lo = lax.bitcast_convert_type(
        x[:, :h].astype(jnp.bfloat16), jnp.uint16).astype(jnp.uint32)
    hi = lax.bitcast_convert_type(
        x[:, h:].astype(jnp.bfloat16), jnp.uint16).astype(jnp.uint32)
    w = lax.bitcast_convert_type(lo | (hi << 16), jnp.int32)
    return jnp.pad(w, ((0, 0), (0, RW - h)))


def kernel(x_user, x_job, edge_label_index):
    E = edge_label_index.shape[1]
    n_win_total = E // W
    n_per_worker = -(-n_win_total // NWORK)
    n_pairs = -(-n_per_worker // 2)
    mesh = plsc.VectorSubcoreMesh(core_axis_name="c", subcore_axis_name="s")
    cp = pltpu.CompilerParams()
    if "needs_layout_passes" in pltpu.CompilerParams.__dataclass_fields__:
        cp = dataclasses.replace(cp, needs_layout_passes=False)
    if "use_tc_tiling_on_sc" in pltpu.CompilerParams.__dataclass_fields__:
        cp = dataclasses.replace(cp, use_tc_tiling_on_sc=False)

    @functools.partial(
        pl.kernel,
        out_type=jax.ShapeDtypeStruct((E,), jnp.float32),
        mesh=mesh,
        compiler_params=cp,
        scratch_types=[
            pltpu.VMEM((W,), jnp.int32),
            pltpu.VMEM((W,), jnp.int32),
            pltpu.VMEM((W,), jnp.int32),
            pltpu.VMEM((W,), jnp.int32),
            pltpu.VMEM((W, RW), jnp.int32),
            pltpu.VMEM((W, RW), jnp.int32),
            pltpu.VMEM((W, RW), jnp.int32),
            pltpu.VMEM((W, RW), jnp.int32),
            pltpu.VMEM((W,), jnp.float32),
            pltpu.SemaphoreType.DMA,
            pltpu.SemaphoreType.DMA,
            pltpu.SemaphoreType.DMA,
            pltpu.SemaphoreType.DMA,
            pltpu.SemaphoreType.DMA,
            pltpu.SemaphoreType.DMA,
            pltpu.SemaphoreType.DMA,
            pltpu.SemaphoreType.DMA,
        ],
    )
    def sc_kernel(xu_hbm, xj_hbm, eu_hbm, ej_hbm, out_hbm,
                  iu0, iu1, ij0, ij1, ru0, ru1, rj0, rj1, out_v,
                  su0, su1, sj0, sj1, tu0, tu1, tj0, tj1):
        wid = lax.axis_index("s") * NC + lax.axis_index("c")
        ius, ijs, rus, rjs = (iu0, iu1), (ij0, ij1), (ru0, ru1), (rj0, rj1)
        sus, sjs = (su0, su1), (sj0, sj1)
        tus, tjs = (tu0, tu1), (tj0, tj1)

        def win_base(k):
            win = wid + k * NWORK
            valid = win < n_win_total
            return jnp.where(valid, win * W, 0), valid

        def start_idx(k, s):
            base, _ = win_base(k)
            pltpu.async_copy(eu_hbm.at[pl.ds(base, W)], ius[s], tus[s])
            pltpu.async_copy(ej_hbm.at[pl.ds(base, W)], ijs[s], tjs[s])

        def start_gathers(k, s):
            base, _ = win_base(k)
            pltpu.make_async_copy(eu_hbm.at[pl.ds(base, W)], ius[s], tus[s]).wait()
            pltpu.make_async_copy(ej_hbm.at[pl.ds(base, W)], ijs[s], tjs[s]).wait()
            pltpu.async_copy(xu_hbm.at[ius[s]], rus[s], sus[s])
            pltpu.async_copy(xj_hbm.at[ijs[s]], rjs[s], sjs[s])

        def wait_rows(s):
            pltpu.make_async_copy(xu_hbm.at[ius[s]], rus[s], sus[s]).wait()
            pltpu.make_async_copy(xj_hbm.at[ijs[s]], rjs[s], sjs[s]).wait()

        def compute_store(k, s):
            ru_v, rj_v = rus[s], rjs[s]

            @pl.loop(0, W // L)
            def _(gi):
                e0 = gi * L
                e_ids = e0 + lax.iota(jnp.int32, L)
                zero = jnp.zeros((L,), jnp.float32)

                def dim_body(t, accs):
                    accs = list(accs)
                    p0 = t * PC
                    for pp in range(PC):
                        pvec = jnp.zeros((L,), jnp.int32) + (p0 + pp)
                        uw = plsc.load_gather(ru_v, [e_ids, pvec])
                        vw = plsc.load_gather(rj_v, [e_ids, pvec])
                        ua, ub = plsc.unpack(
                            plsc.bitcast(uw, jnp.bfloat16),
                            format=plsc.PackFormat.INTERLEAVED)
                        va, vb = plsc.unpack(
                            plsc.bitcast(vw, jnp.bfloat16),
                            format=plsc.PackFormat.INTERLEAVED)
                        accs[(2 * pp) % 4] = accs[(2 * pp) % 4] + ua * va
                        accs[(2 * pp + 1) % 4] = accs[(2 * pp + 1) % 4] + ub * vb
                    return tuple(accs)

                a0, a1, a2, a3 = lax.fori_loop(
                    0, DP // PC, dim_body, (zero, zero, zero, zero))
                out_v[pl.ds(e0, L)] = (a0 + a1) + (a2 + a3)

            base, valid = win_base(k)

            @pl.when(valid)
            def _():
                pltpu.sync_copy(out_v, out_hbm.at[pl.ds(base, W)])

        start_idx(0, 0)
        start_gathers(0, 0)
        start_idx(1, 1)

        @pl.loop(0, n_pairs)
        def _(i):
            k0 = 2 * i
            start_gathers(k0 + 1, 1)
            wait_rows(0)
            start_idx(k0 + 2, 0)
            compute_store(k0, 0)
            start_gathers(k0 + 2, 0)
            wait_rows(1)
            start_idx(k0 + 3, 1)
            compute_store(k0 + 1, 1)

        wait_rows(0)
        base, _ = win_base(2 * n_pairs + 1)
        pltpu.make_async_copy(eu_hbm.at[pl.ds(base, W)], ius[1], tus[1]).wait()
        pltpu.make_async_copy(ej_hbm.at[pl.ds(base, W)], ijs[1], tjs[1]).wait()

    return sc_kernel(_pack_bf16(x_user), _pack_bf16(x_job),
                     edge_label_index[0], edge_label_index[1])


# speedup vs baseline: 3.9583x; 1.1574x over previous
import dataclasses
import functools

import jax
import jax.numpy as jnp
from jax import lax
from jax.experimental import pallas as pl
from jax.experimental.pallas import tpu as pltpu
from jax.experimental.pallas import tpu_sc as plsc

D = 256
DP = D // 2
RW = DP + 1
L = 16
NC, NS = 2, 16
NWORK = NC * NS
W = 128
PC = 8


def _pack_bf16(x):
    n, d = x.shape
    h = d // 2
    lo = lax.bitcast_convert_type(
        x[:, :h].astype(jnp.bfloat16), jnp.uint16).astype(jnp.uint32)
    hi = lax.bitcast_convert_type(
        x[:, h:].astype(jnp.bfloat16), jnp.uint16).astype(jnp.uint32)
    w = lax.bitcast_convert_type(lo | (hi << 16), jnp.int32)
    return jnp.pad(w, ((0, 0), (0, RW - h)))


def kernel(x_user, x_job, edge_label_index):
    E = edge_label_index.shape[1]
    n_win_total = E // W
    n_per_worker = -(-n_win_total // NWORK)
    n_pairs = -(-n_per_worker // 2)
    mesh = plsc.VectorSubcoreMesh(core_axis_name="c", subcore_axis_name="s")
    cp = pltpu.CompilerParams()
    if "needs_layout_passes" in pltpu.CompilerParams.__dataclass_fields__:
        cp = dataclasses.replace(cp, needs_layout_passes=False)
    if "use_tc_tiling_on_sc" in pltpu.CompilerParams.__dataclass_fields__:
        cp = dataclasses.replace(cp, use_tc_tiling_on_sc=False)

    @functools.partial(
        pl.kernel,
        out_type=jax.ShapeDtypeStruct((E,), jnp.float32),
        mesh=mesh,
        compiler_params=cp,
        scratch_types=[
            pltpu.VMEM((W,), jnp.int32),
            pltpu.VMEM((W,), jnp.int32),
            pltpu.VMEM((W,), jnp.int32),
            pltpu.VMEM((W,), jnp.int32),
            pltpu.VMEM((W, RW), jnp.int32),
            pltpu.VMEM((W, RW), jnp.int32),
            pltpu.VMEM((W, RW), jnp.int32),
            pltpu.VMEM((W, RW), jnp.int32),
            pltpu.VMEM((W,), jnp.float32),
            pltpu.SemaphoreType.DMA,
            pltpu.SemaphoreType.DMA,
            pltpu.SemaphoreType.DMA,
            pltpu.SemaphoreType.DMA,
            pltpu.SemaphoreType.DMA,
            pltpu.SemaphoreType.DMA,
            pltpu.SemaphoreType.DMA,
            pltpu.SemaphoreType.DMA,
        ],
    )
    def sc_kernel(xu_hbm, xj_hbm, eu_hbm, ej_hbm, out_hbm,
                  iu0, iu1, ij0, ij1, ru0, ru1, rj0, rj1, out_v,
                  su0, su1, sj0, sj1, tu0, tu1, tj0, tj1):
        wid = lax.axis_index("s") * NC + lax.axis_index("c")
        ius, ijs, rus, rjs = (iu0, iu1), (ij0, ij1), (ru0, ru1), (rj0, rj1)
        sus, sjs = (su0, su1), (sj0, sj1)
        tus, tjs = (tu0, tu1), (tj0, tj1)

        def win_base(k):
            win = wid + k * NWORK
            valid = win < n_win_total
            return jnp.where(valid, win * W, 0), valid

        def start_idx(k, s):
            base, _ = win_base(k)
            pltpu.async_copy(eu_hbm.at[pl.ds(base, W)], ius[s], tus[s])
            pltpu.async_copy(ej_hbm.at[pl.ds(base, W)], ijs[s], tjs[s])

        def start_gathers(k, s):
            base, _ = win_base(k)
            pltpu.make_async_copy(eu_hbm.at[pl.ds(base, W)], ius[s], tus[s]).wait()
            pltpu.make_async_copy(ej_hbm.at[pl.ds(base, W)], ijs[s], tjs[s]).wait()
            pltpu.async_copy(xu_hbm.at[ius[s]], rus[s], sus[s])
            pltpu.async_copy(xj_hbm.at[ijs[s]], rjs[s], sjs[s])

        def wait_rows(s):
            pltpu.make_async_copy(xu_hbm.at[ius[s]], rus[s], sus[s]).wait()
            pltpu.make_async_copy(xj_hbm.at[ijs[s]], rjs[s], sjs[s]).wait()

        def compute_store(k, s):
            ru_v, rj_v = rus[s], rjs[s]

            @pl.loop(0, W // L)
            def _(gi):
                e0 = gi * L
                e_ids = e0 + lax.iota(jnp.int32, L)
                zero = jnp.zeros((L,), jnp.float32)

                def dim_body(t, accs):
                    accs = list(accs)
                    p0 = t * PC
                    for pp in range(PC):
                        pvec = jnp.zeros((L,), jnp.int32) + (p0 + pp)
                        uw = plsc.load_gather(ru_v, [e_ids, pvec])
                        vw = plsc.load_gather(rj_v, [e_ids, pvec])
                        prod = (plsc.bitcast(uw, jnp.bfloat16)
                                * plsc.bitcast(vw, jnp.bfloat16))
                        pa, pb = plsc.unpack(
                            prod, format=plsc.PackFormat.INTERLEAVED)
                        accs[(2 * pp) % 4] = accs[(2 * pp) % 4] + pa
                        accs[(2 * pp + 1) % 4] = accs[(2 * pp + 1) % 4] + pb
                    return tuple(accs)

                a0, a1, a2, a3 = lax.fori_loop(
                    0, DP // PC, dim_body, (zero, zero, zero, zero))
                out_v[pl.ds(e0, L)] = (a0 + a1) + (a2 + a3)

            base, valid = win_base(k)

            @pl.when(valid)
            def _():
                pltpu.sync_copy(out_v, out_hbm.at[pl.ds(base, W)])

        start_idx(0, 0)
        start_gathers(0, 0)
        start_idx(1, 1)

        @pl.loop(0, n_pairs)
        def _(i):
            k0 = 2 * i
            start_gathers(k0 + 1, 1)
            wait_rows(0)
            start_idx(k0 + 2, 0)
            compute_store(k0, 0)
            start_gathers(k0 + 2, 0)
            wait_rows(1)
            start_idx(k0 + 3, 1)
            compute_store(k0 + 1, 1)

        wait_rows(0)
        base, _ = win_base(2 * n_pairs + 1)
        pltpu.make_async_copy(eu_hbm.at[pl.ds(base, W)], ius[1], tus[1]).wait()
        pltpu.make_async_copy(ej_hbm.at[pl.ds(base, W)], ijs[1], tjs[1]).wait()

    return sc_kernel(_pack_bf16(x_user), _pack_bf16(x_job),
                     edge_label_index[0], edge_label_index[1])
